# baseline (speedup 1.0000x reference)
import functools

import jax
import jax.numpy as jnp
from jax import lax
from jax.experimental import pallas as pl
from jax.experimental.pallas import tpu as pltpu
from jax.experimental.pallas import tpu_sc as plsc

N = 10000
F = 128
E = 320000
NC = 2
NS = 16
NW = NC * NS
KC = 80
CPT = 125
EPT = KC * CPT
NP = 10240
RPSW = NP // NS
NB = 2
PSLOT = 6
PLEAD = 4

_MESH = plsc.VectorSubcoreMesh(
    core_axis_name="c", subcore_axis_name="s", num_cores=NC, num_subcores=NS)


def _prop_pass(hs, src, dst, ew, out, srcb, dstb, ewb,
               gb0, gb1, sb0, sb1, gsem0, gsem1, ssem0, ssem1,
               isem0, isem1, shared):
    c = lax.axis_index("c")
    s = lax.axis_index("s")
    wid = c * NS + s
    gb = (gb0, gb1)
    sb = (sb0, sb1)
    gsem = (gsem0, gsem1)
    ssem = (ssem0, ssem1)
    isem = (isem0, isem1)
    zvec = jnp.zeros((16,), jnp.float32)
    ebase = wid * EPT

    def idx_desc(i, slot, sem):
        sl = pl.ds(ebase + i * KC, KC)
        return (pltpu.make_async_copy(src.at[sl], srcb.at[slot], sem),
                pltpu.make_async_copy(dst.at[sl], dstb.at[slot], sem),
                pltpu.make_async_copy(ew.at[sl], ewb.at[slot], sem))

    HK = KC // 2

    def gather_descs(q2, gbuf, sem):
        return (pltpu.make_async_copy(hs.at[srcb.at[q2, pl.ds(0, HK)]],
                                      gbuf.at[pl.ds(0, HK)], sem),
                pltpu.make_async_copy(hs.at[srcb.at[q2, pl.ds(HK, HK)]],
                                      gbuf.at[pl.ds(HK, HK)], sem))

    for i in range(PLEAD):
        for d in idx_desc(i, i, isem[i % 2]):
            d.start()
    for i in range(NB):
        for d in idx_desc(i, i, isem[i % 2]):
            d.wait()
    for d in gather_descs(0, gb0, gsem0):
        d.start()
    for d in gather_descs(1, gb1, gsem1):
        d.start()

    def zfill(r, carry):
        for j in range(F // 16):
            sb0[r, pl.ds(j * 16, 16)] = zvec
        return carry
    lax.fori_loop(0, KC, zfill, 0)
    zcps = [pltpu.make_async_copy(
        sb0, shared.at[pl.ds(s * RPSW + t * KC, KC)], isem0)
        for t in range(RPSW // KC)]
    for d in zcps:
        d.start()
    for d in zcps:
        d.wait()
    plsc.subcore_barrier()

    def run_chunk(i, b):
        q = lax.rem(i, PSLOT)
        gbuf = gb[b]
        sbuf = sb[b]
        for d in gather_descs(q, gbuf, gsem[b]):
            d.wait()

        @pl.when(i >= NB)
        def _():
            qm2 = lax.rem(i + (PSLOT - 2), PSLOT)
            pltpu.make_async_copy(
                sbuf, shared.at[dstb.at[qm2]], ssem[b]).wait()

        @plsc.parallel_loop(0, KC // 16, 1, unroll=2)
        def _scale(g):
            cvec = ewb[q, pl.ds(g * 16, 16)]
            for l in range(16):
                e = g * 16 + l
                cf = cvec[l]
                for j in range(F // 16):
                    sl = pl.ds(j * 16, 16)
                    sbuf[e, sl] = gbuf[e, sl] * cf

        @pl.when(i + NB < CPT)
        def _():
            q2 = lax.rem(i + 2, PSLOT)
            for d in idx_desc(i + 2, q2, isem[b]):
                d.wait()
            for d in gather_descs(q2, gbuf, gsem[b]):
                d.start()

        pltpu.async_copy(sbuf, shared.at[dstb.at[q]], ssem[b], add=True)

        @pl.when(i + PLEAD < CPT)
        def _():
            q4 = lax.rem(i + PLEAD, PSLOT)
            for d in idx_desc(i + PLEAD, q4, isem[b]):
                d.start()

    def chunk_pair(ii, carry):
        i2 = ii * 2
        for b in range(NB):
            run_chunk(i2 + b, b)
        return carry
    lax.fori_loop(0, CPT // 2, chunk_pair, 0)
    run_chunk(jnp.int32(CPT - 1), (CPT - 1) % 2)

    for i in range(CPT - NB, CPT):
        pltpu.make_async_copy(
            sb[i % 2], shared.at[dstb.at[i % PSLOT]], ssem[i % 2]).wait()
    plsc.subcore_barrier()

    pltpu.sync_copy(shared.at[pl.ds(s * RPSW, RPSW)],
                    out.at[c, pl.ds(s * RPSW, RPSW)])


def _deg_body(dst, ew, out, dstb, ewb, sb0, sb1,
              ssem0, ssem1, isem0, isem1, shared):
    c = lax.axis_index("c")
    s = lax.axis_index("s")
    wid = c * NS + s
    sb = (sb0, sb1)
    ssem = (ssem0, ssem1)
    isem = (isem0, isem1)
    zvec = jnp.zeros((16,), jnp.float32)
    ebase = wid * EPT

    def idx_desc(i, slot, sem):
        sl = pl.ds(ebase + i * KC, KC)
        return (pltpu.make_async_copy(dst.at[sl], dstb.at[slot], sem),
                pltpu.make_async_copy(ew.at[sl], ewb.at[slot], sem))

    for i in range(PLEAD):
        for d in idx_desc(i, i, isem[i % 2]):
            d.start()

    def zfill(r, carry):
        for j in range(F // 16):
            sb0[r, pl.ds(j * 16, 16)] = zvec
            sb1[r, pl.ds(j * 16, 16)] = zvec
        return carry
    lax.fori_loop(0, KC, zfill, 0)
    zcps = [pltpu.make_async_copy(
        sb0, shared.at[pl.ds(s * RPSW + t * KC, KC)], ssem0)
        for t in range(RPSW // KC)]
    for d in zcps:
        d.start()
    for d in zcps:
        d.wait()
    plsc.subcore_barrier()

    def run_chunk(i, b):
        q = lax.rem(i, PSLOT)
        sbuf = sb[b]

        @pl.when(i >= NB)
        def _():
            qm2 = lax.rem(i + (PSLOT - 2), PSLOT)
            pltpu.make_async_copy(
                sbuf, shared.at[dstb.at[qm2]], ssem[b]).wait()

        for d in idx_desc(i, q, isem[b]):
            d.wait()

        @plsc.parallel_loop(0, KC // 16, 1, unroll=2)
        def _fill(g):
            cvec = ewb[q, pl.ds(g * 16, 16)]
            for l in range(16):
                sbuf[g * 16 + l, pl.ds(0, 16)] = jnp.broadcast_to(
                    cvec[l], (16,))

        pltpu.async_copy(sbuf, shared.at[dstb.at[q]], ssem[b], add=True)

        @pl.when(i + PLEAD < CPT)
        def _():
            q4 = lax.rem(i + PLEAD, PSLOT)
            for d in idx_desc(i + PLEAD, q4, isem[b]):
                d.start()

    def chunk_pair(ii, carry):
        i2 = ii * 2
        for b in range(NB):
            run_chunk(i2 + b, b)
        return carry
    lax.fori_loop(0, CPT // 2, chunk_pair, 0)
    run_chunk(jnp.int32(CPT - 1), (CPT - 1) % 2)

    for i in range(CPT - NB, CPT):
        pltpu.make_async_copy(
            sb[i % 2], shared.at[dstb.at[i % PSLOT]], ssem[i % 2]).wait()
    plsc.subcore_barrier()

    pltpu.sync_copy(shared.at[pl.ds(s * RPSW, RPSW)],
                    out.at[c, pl.ds(s * RPSW, RPSW)])


_deg_kernel = pl.kernel(
    _deg_body,
    out_type=jax.ShapeDtypeStruct((NC, NP, F), jnp.float32),
    mesh=_MESH,
    scratch_types=[
        pltpu.VMEM((PSLOT, KC), jnp.int32),
        pltpu.VMEM((PSLOT, KC), jnp.float32),
        pltpu.VMEM((KC, F), jnp.float32),
        pltpu.VMEM((KC, F), jnp.float32),
        pltpu.SemaphoreType.DMA,
        pltpu.SemaphoreType.DMA,
        pltpu.SemaphoreType.DMA,
        pltpu.SemaphoreType.DMA,
        pltpu.VMEM_SHARED((NP, F), jnp.float32),
    ],
)


def _prop_body(hs, src, dst, ew, out, *scr):
    _prop_pass(hs, src, dst, ew, out, *scr)


def _prop2_body(hsA, hsB, src, dst, ew, outA, outB, *scr):
    _prop_pass(hsA, src, dst, ew, outA, *scr)
    _prop_pass(hsB, src, dst, ew, outB, *scr)


_PROP_SCRATCH = [
        pltpu.VMEM((PSLOT, KC), jnp.int32),
        pltpu.VMEM((PSLOT, KC), jnp.int32),
        pltpu.VMEM((PSLOT, KC), jnp.float32),
        pltpu.VMEM((KC, F), jnp.float32),
        pltpu.VMEM((KC, F), jnp.float32),
        pltpu.VMEM((KC, F), jnp.float32),
        pltpu.VMEM((KC, F), jnp.float32),
        pltpu.SemaphoreType.DMA,
        pltpu.SemaphoreType.DMA,
        pltpu.SemaphoreType.DMA,
        pltpu.SemaphoreType.DMA,
        pltpu.SemaphoreType.DMA,
        pltpu.SemaphoreType.DMA,
        pltpu.VMEM_SHARED((NP, F), jnp.float32),
]

_OUT_T = jax.ShapeDtypeStruct((NC, NP, F), jnp.float32)

_prop_kernel = pl.kernel(
    _prop_body,
    out_type=_OUT_T,
    mesh=_MESH,
    scratch_types=list(_PROP_SCRATCH),
)

_prop2_kernel = pl.kernel(
    _prop2_body,
    out_type=(_OUT_T, _OUT_T),
    mesh=_MESH,
    scratch_types=list(_PROP_SCRATCH),
)


BR = 1000
GR = N // BR


def _dinv_body(dg_ref, o_ref):
    deg = dg_ref[0, :, 0:1] + dg_ref[1, :, 0:1] + 1.0
    o_ref[...] = jnp.where(deg > 0, lax.rsqrt(deg), 0.0)


def _tc_dinv(degp):
    return pl.pallas_call(
        _dinv_body,
        grid=(GR,),
        in_specs=[pl.BlockSpec((NC, BR, F), lambda i: (0, i, 0))],
        out_specs=pl.BlockSpec((BR, 1), lambda i: (i, 0)),
        out_shape=jax.ShapeDtypeStruct((N, 1), jnp.float32),
    )(degp)


def _mm_scale_body(x_ref, w_ref, d_ref, o_ref):
    o_ref[...] = jnp.dot(x_ref[...], w_ref[...],
                         preferred_element_type=jnp.float32) * d_ref[...]


def _tc_mm_scale(x, w, dinv):
    return pl.pallas_call(
        _mm_scale_body,
        grid=(GR,),
        in_specs=[
            pl.BlockSpec((BR, F), lambda i: (i, 0)),
            pl.BlockSpec((F, F), lambda i: (0, 0)),
            pl.BlockSpec((BR, 1), lambda i: (i, 0)),
        ],
        out_specs=pl.BlockSpec((BR, F), lambda i: (i, 0)),
        out_shape=jax.ShapeDtypeStruct((N, F), jnp.float32),
    )(x, w, dinv)


def _combine_mm_body(g_ref, hs_ref, b_ref, d_ref, w_ref, o_ref):
    t = (g_ref[0] + g_ref[1] + hs_ref[...]) * d_ref[...] + b_ref[...]
    t = jnp.maximum(t, 0.0)
    o_ref[...] = jnp.dot(t, w_ref[...],
                         preferred_element_type=jnp.float32) * d_ref[...]


def _tc_combine_mm(g, hs, b, w, dinv):
    return pl.pallas_call(
        _combine_mm_body,
        grid=(GR,),
        in_specs=[
            pl.BlockSpec((NC, BR, F), lambda i: (0, i, 0)),
            pl.BlockSpec((BR, F), lambda i: (i, 0)),
            pl.BlockSpec((1, F), lambda i: (0, 0)),
            pl.BlockSpec((BR, 1), lambda i: (i, 0)),
            pl.BlockSpec((F, F), lambda i: (0, 0)),
        ],
        out_specs=pl.BlockSpec((BR, F), lambda i: (i, 0)),
        out_shape=jax.ShapeDtypeStruct((N, F), jnp.float32),
    )(g, hs, b.reshape(1, F), dinv, w)


def _combine_body(g_ref, hs_ref, b_ref, d_ref, o_ref, *, relu):
    t = (g_ref[0] + g_ref[1] + hs_ref[...]) * d_ref[...] + b_ref[...]
    o_ref[...] = jnp.maximum(t, 0.0) if relu else t


def _tc_combine(g, hs, b, dinv, relu):
    nf = hs.shape[1]
    return pl.pallas_call(
        functools.partial(_combine_body, relu=relu),
        grid=(GR,),
        in_specs=[
            pl.BlockSpec((NC, BR, nf), lambda i: (0, i, 0)),
            pl.BlockSpec((BR, nf), lambda i: (i, 0)),
            pl.BlockSpec((1, nf), lambda i: (0, 0)),
            pl.BlockSpec((BR, 1), lambda i: (i, 0)),
        ],
        out_specs=pl.BlockSpec((BR, nf), lambda i: (i, 0)),
        out_shape=jax.ShapeDtypeStruct((N, nf), jnp.float32),
    )(g, hs, b.reshape(1, nf), dinv)


def _nt_body(a_ref, b_ref, o_ref):
    o_ref[...] = lax.dot_general(
        a_ref[...], b_ref[...], (((1,), (1,)), ((), ())),
        preferred_element_type=jnp.float32)


def _tc_matmul_nt(s):
    br, bc = 1024, 2048
    gi = -(-N // br)
    gj = -(-N // bc)
    return pl.pallas_call(
        _nt_body,
        grid=(gi, gj),
        in_specs=[
            pl.BlockSpec((br, F), lambda i, j: (i, 0)),
            pl.BlockSpec((bc, F), lambda i, j: (j, 0)),
        ],
        out_specs=pl.BlockSpec((br, bc), lambda i, j: (i, j)),
        out_shape=jax.ShapeDtypeStruct((N, N), jnp.float32),
    )(s, s)


def kernel(x, edge_index, edge_weight, W1e, b1e, W2e, b2e,
           W1a, b1a, W2a, b2a, W1s, b1s):
    src2 = edge_index[0]
    dst2 = edge_index[1]
    ew2 = edge_weight

    degp = _deg_kernel(dst2, ew2)
    dinv = _tc_dinv(degp)

    hs1 = _tc_mm_scale(x, W1e, dinv)
    g1 = _prop_kernel(hs1, src2, dst2, ew2)
    hs2 = _tc_combine_mm(g1, hs1, b1e, W2e, dinv)
    g2 = _prop_kernel(hs2, src2, dst2, ew2)
    x_encoded = _tc_combine(g2, hs2, b2e, dinv, relu=True)

    hs5 = _tc_mm_scale(x_encoded, W1s, dinv)
    hs3 = _tc_mm_scale(x_encoded, W1a, dinv)
    g5, g3 = _prop2_kernel(hs5, hs3, src2, dst2, ew2)
    s = _tc_combine(g5, hs5, b1s, dinv, relu=True)
    struct_reconstructed = _tc_matmul_nt(s)

    hs4 = _tc_combine_mm(g3, hs3, b1a, W2a, dinv)
    g4 = _prop_kernel(hs4, src2, dst2, ew2)
    x_hat = _tc_combine(g4, hs4, b2a, dinv, relu=False)

    return (struct_reconstructed, x_hat, x_encoded)


# speedup vs baseline: 1.0177x; 1.0177x over previous
import functools

import jax
import jax.numpy as jnp
from jax import lax
from jax.experimental import pallas as pl
from jax.experimental.pallas import tpu as pltpu
from jax.experimental.pallas import tpu_sc as plsc

N = 10000
F = 128
E = 320000
NC = 2
NS = 16
NW = NC * NS
KC = 80
CPT = 125
EPT = KC * CPT
NP = 10240
RPSW = NP // NS
NB = 2
PSLOT = 6
PLEAD = 4

_MESH = plsc.VectorSubcoreMesh(
    core_axis_name="c", subcore_axis_name="s", num_cores=NC, num_subcores=NS)


def _prop_body(hs, src, dst, ew, out, srcb, dstb, ewb,
               gb0, gb1, sb0, sb1, gsem0, gsem1, ssem0, ssem1,
               isem0, isem1, shared):
    c = lax.axis_index("c")
    s = lax.axis_index("s")
    wid = c * NS + s
    gb = (gb0, gb1)
    sb = (sb0, sb1)
    gsem = (gsem0, gsem1)
    ssem = (ssem0, ssem1)
    isem = (isem0, isem1)
    zvec = jnp.zeros((16,), jnp.float32)
    ebase = wid * EPT

    def idx_desc(i, slot, sem):
        sl = pl.ds(ebase + i * KC, KC)
        return (pltpu.make_async_copy(src.at[sl], srcb.at[slot], sem),
                pltpu.make_async_copy(dst.at[sl], dstb.at[slot], sem),
                pltpu.make_async_copy(ew.at[sl], ewb.at[slot], sem))

    HK = KC // 2

    def gather_descs(q2, gbuf, sem):
        return (pltpu.make_async_copy(hs.at[srcb.at[q2, pl.ds(0, HK)]],
                                      gbuf.at[pl.ds(0, HK)], sem),
                pltpu.make_async_copy(hs.at[srcb.at[q2, pl.ds(HK, HK)]],
                                      gbuf.at[pl.ds(HK, HK)], sem))

    for i in range(PLEAD):
        for d in idx_desc(i, i, isem[i % 2]):
            d.start()
    for i in range(NB):
        for d in idx_desc(i, i, isem[i % 2]):
            d.wait()
    for d in gather_descs(0, gb0, gsem0):
        d.start()
    for d in gather_descs(1, gb1, gsem1):
        d.start()

    def zfill(r, carry):
        for j in range(F // 16):
            sb0[r, pl.ds(j * 16, 16)] = zvec
        return carry
    lax.fori_loop(0, KC, zfill, 0)
    zcps = [pltpu.make_async_copy(
        sb0, shared.at[pl.ds(s * RPSW + t * KC, KC)], isem0)
        for t in range(RPSW // KC)]
    for d in zcps:
        d.start()
    for d in zcps:
        d.wait()
    plsc.subcore_barrier()

    def run_chunk(i, b):
        q = lax.rem(i, PSLOT)
        gbuf = gb[b]
        sbuf = sb[b]
        for d in gather_descs(q, gbuf, gsem[b]):
            d.wait()

        @pl.when(i >= NB)
        def _():
            qm2 = lax.rem(i + (PSLOT - 2), PSLOT)
            pltpu.make_async_copy(
                sbuf, shared.at[dstb.at[qm2]], ssem[b]).wait()

        @plsc.parallel_loop(0, KC // 16, 1, unroll=2)
        def _scale(g):
            cvec = ewb[q, pl.ds(g * 16, 16)]
            for l in range(16):
                e = g * 16 + l
                cf = cvec[l]
                for j in range(F // 16):
                    sl = pl.ds(j * 16, 16)
                    sbuf[e, sl] = gbuf[e, sl] * cf

        @pl.when(i + NB < CPT)
        def _():
            q2 = lax.rem(i + 2, PSLOT)
            for d in idx_desc(i + 2, q2, isem[b]):
                d.wait()
            for d in gather_descs(q2, gbuf, gsem[b]):
                d.start()

        pltpu.async_copy(sbuf, shared.at[dstb.at[q]], ssem[b], add=True)

        @pl.when(i + PLEAD < CPT)
        def _():
            q4 = lax.rem(i + PLEAD, PSLOT)
            for d in idx_desc(i + PLEAD, q4, isem[b]):
                d.start()

    def chunk_pair(ii, carry):
        i2 = ii * 2
        for b in range(NB):
            run_chunk(i2 + b, b)
        return carry
    lax.fori_loop(0, CPT // 2, chunk_pair, 0)
    run_chunk(jnp.int32(CPT - 1), (CPT - 1) % 2)

    for i in range(CPT - NB, CPT):
        pltpu.make_async_copy(
            sb[i % 2], shared.at[dstb.at[i % PSLOT]], ssem[i % 2]).wait()
    plsc.subcore_barrier()

    pltpu.sync_copy(shared.at[pl.ds(s * RPSW, RPSW)],
                    out.at[c, pl.ds(s * RPSW, RPSW)])


def _deg_body(dst, ew, out, dstb, ewb, sb0, sb1,
              ssem0, ssem1, isem0, isem1, shared):
    c = lax.axis_index("c")
    s = lax.axis_index("s")
    wid = c * NS + s
    sb = (sb0, sb1)
    ssem = (ssem0, ssem1)
    isem = (isem0, isem1)
    zvec = jnp.zeros((16,), jnp.float32)
    ebase = wid * EPT

    def idx_desc(i, slot, sem):
        sl = pl.ds(ebase + i * KC, KC)
        return (pltpu.make_async_copy(dst.at[sl], dstb.at[slot], sem),
                pltpu.make_async_copy(ew.at[sl], ewb.at[slot], sem))

    for i in range(PLEAD):
        for d in idx_desc(i, i, isem[i % 2]):
            d.start()

    def zfill(r, carry):
        for j in range(F // 16):
            sb0[r, pl.ds(j * 16, 16)] = zvec
            sb1[r, pl.ds(j * 16, 16)] = zvec
        return carry
    lax.fori_loop(0, KC, zfill, 0)
    zcps = [pltpu.make_async_copy(
        sb0, shared.at[pl.ds(s * RPSW + t * KC, KC)], ssem0)
        for t in range(RPSW // KC)]
    for d in zcps:
        d.start()
    for d in zcps:
        d.wait()
    plsc.subcore_barrier()

    def run_chunk(i, b):
        q = lax.rem(i, PSLOT)
        sbuf = sb[b]

        @pl.when(i >= NB)
        def _():
            qm2 = lax.rem(i + (PSLOT - 2), PSLOT)
            pltpu.make_async_copy(
                sbuf, shared.at[dstb.at[qm2]], ssem[b]).wait()

        for d in idx_desc(i, q, isem[b]):
            d.wait()

        @plsc.parallel_loop(0, KC // 16, 1, unroll=2)
        def _fill(g):
            cvec = ewb[q, pl.ds(g * 16, 16)]
            for l in range(16):
                sbuf[g * 16 + l, pl.ds(0, 16)] = jnp.broadcast_to(
                    cvec[l], (16,))

        pltpu.async_copy(sbuf, shared.at[dstb.at[q]], ssem[b], add=True)

        @pl.when(i + PLEAD < CPT)
        def _():
            q4 = lax.rem(i + PLEAD, PSLOT)
            for d in idx_desc(i + PLEAD, q4, isem[b]):
                d.start()

    def chunk_pair(ii, carry):
        i2 = ii * 2
        for b in range(NB):
            run_chunk(i2 + b, b)
        return carry
    lax.fori_loop(0, CPT // 2, chunk_pair, 0)
    run_chunk(jnp.int32(CPT - 1), (CPT - 1) % 2)

    for i in range(CPT - NB, CPT):
        pltpu.make_async_copy(
            sb[i % 2], shared.at[dstb.at[i % PSLOT]], ssem[i % 2]).wait()
    plsc.subcore_barrier()

    pltpu.sync_copy(shared.at[pl.ds(s * RPSW, RPSW)],
                    out.at[c, pl.ds(s * RPSW, RPSW)])


_deg_kernel = pl.kernel(
    _deg_body,
    out_type=jax.ShapeDtypeStruct((NC, NP, F), jnp.float32),
    mesh=_MESH,
    scratch_types=[
        pltpu.VMEM((PSLOT, KC), jnp.int32),
        pltpu.VMEM((PSLOT, KC), jnp.float32),
        pltpu.VMEM((KC, F), jnp.float32),
        pltpu.VMEM((KC, F), jnp.float32),
        pltpu.SemaphoreType.DMA,
        pltpu.SemaphoreType.DMA,
        pltpu.SemaphoreType.DMA,
        pltpu.SemaphoreType.DMA,
        pltpu.VMEM_SHARED((NP, F), jnp.float32),
    ],
)


_prop_kernel = pl.kernel(
    _prop_body,
    out_type=jax.ShapeDtypeStruct((NC, NP, F), jnp.float32),
    mesh=_MESH,
    scratch_types=[
        pltpu.VMEM((PSLOT, KC), jnp.int32),
        pltpu.VMEM((PSLOT, KC), jnp.int32),
        pltpu.VMEM((PSLOT, KC), jnp.float32),
        pltpu.VMEM((KC, F), jnp.float32),
        pltpu.VMEM((KC, F), jnp.float32),
        pltpu.VMEM((KC, F), jnp.float32),
        pltpu.VMEM((KC, F), jnp.float32),
        pltpu.SemaphoreType.DMA,
        pltpu.SemaphoreType.DMA,
        pltpu.SemaphoreType.DMA,
        pltpu.SemaphoreType.DMA,
        pltpu.SemaphoreType.DMA,
        pltpu.SemaphoreType.DMA,
        pltpu.VMEM_SHARED((NP, F), jnp.float32),
    ],
)


BR = 1000
GR = N // BR


def _dinv_body(dg_ref, o_ref):
    deg = dg_ref[0, :, 0:1] + dg_ref[1, :, 0:1] + 1.0
    o_ref[...] = jnp.where(deg > 0, lax.rsqrt(deg), 0.0)


def _tc_dinv(degp):
    return pl.pallas_call(
        _dinv_body,
        grid=(GR,),
        in_specs=[pl.BlockSpec((NC, BR, F), lambda i: (0, i, 0))],
        out_specs=pl.BlockSpec((BR, 1), lambda i: (i, 0)),
        out_shape=jax.ShapeDtypeStruct((N, 1), jnp.float32),
    )(degp)


def _mm_scale_body(x_ref, w_ref, d_ref, o_ref):
    o_ref[...] = jnp.dot(x_ref[...], w_ref[...],
                         preferred_element_type=jnp.float32) * d_ref[...]


def _tc_mm_scale(x, w, dinv):
    return pl.pallas_call(
        _mm_scale_body,
        grid=(GR,),
        in_specs=[
            pl.BlockSpec((BR, F), lambda i: (i, 0)),
            pl.BlockSpec((F, F), lambda i: (0, 0)),
            pl.BlockSpec((BR, 1), lambda i: (i, 0)),
        ],
        out_specs=pl.BlockSpec((BR, F), lambda i: (i, 0)),
        out_shape=jax.ShapeDtypeStruct((N, F), jnp.float32),
    )(x, w, dinv)


def _combine_mm_body(g_ref, hs_ref, b_ref, d_ref, w_ref, o_ref):
    t = (g_ref[0] + g_ref[1] + hs_ref[...]) * d_ref[...] + b_ref[...]
    t = jnp.maximum(t, 0.0)
    o_ref[...] = jnp.dot(t, w_ref[...],
                         preferred_element_type=jnp.float32) * d_ref[...]


def _tc_combine_mm(g, hs, b, w, dinv):
    return pl.pallas_call(
        _combine_mm_body,
        grid=(GR,),
        in_specs=[
            pl.BlockSpec((NC, BR, F), lambda i: (0, i, 0)),
            pl.BlockSpec((BR, F), lambda i: (i, 0)),
            pl.BlockSpec((1, F), lambda i: (0, 0)),
            pl.BlockSpec((BR, 1), lambda i: (i, 0)),
            pl.BlockSpec((F, F), lambda i: (0, 0)),
        ],
        out_specs=pl.BlockSpec((BR, F), lambda i: (i, 0)),
        out_shape=jax.ShapeDtypeStruct((N, F), jnp.float32),
    )(g, hs, b.reshape(1, F), dinv, w)


def _combine_body(g_ref, hs_ref, b_ref, d_ref, o_ref, *, relu):
    t = (g_ref[0] + g_ref[1] + hs_ref[...]) * d_ref[...] + b_ref[...]
    o_ref[...] = jnp.maximum(t, 0.0) if relu else t


def _tc_combine(g, hs, b, dinv, relu):
    nf = hs.shape[1]
    return pl.pallas_call(
        functools.partial(_combine_body, relu=relu),
        grid=(GR,),
        in_specs=[
            pl.BlockSpec((NC, BR, nf), lambda i: (0, i, 0)),
            pl.BlockSpec((BR, nf), lambda i: (i, 0)),
            pl.BlockSpec((1, nf), lambda i: (0, 0)),
            pl.BlockSpec((BR, 1), lambda i: (i, 0)),
        ],
        out_specs=pl.BlockSpec((BR, nf), lambda i: (i, 0)),
        out_shape=jax.ShapeDtypeStruct((N, nf), jnp.float32),
    )(g, hs, b.reshape(1, nf), dinv)


def _nt_body(a_ref, b_ref, o_ref):
    o_ref[...] = lax.dot_general(
        a_ref[...], b_ref[...], (((1,), (1,)), ((), ())),
        preferred_element_type=jnp.float32)


def _tc_matmul_nt(s):
    br, bc = 1024, 2048
    gi = -(-N // br)
    gj = -(-N // bc)
    return pl.pallas_call(
        _nt_body,
        grid=(gi, gj),
        in_specs=[
            pl.BlockSpec((br, F), lambda i, j: (i, 0)),
            pl.BlockSpec((bc, F), lambda i, j: (j, 0)),
        ],
        out_specs=pl.BlockSpec((br, bc), lambda i, j: (i, j)),
        out_shape=jax.ShapeDtypeStruct((N, N), jnp.float32),
    )(s, s)


def kernel(x, edge_index, edge_weight, W1e, b1e, W2e, b2e,
           W1a, b1a, W2a, b2a, W1s, b1s):
    src2 = edge_index[0]
    dst2 = edge_index[1]
    ew2 = edge_weight

    degp = _deg_kernel(dst2, ew2)
    dinv = _tc_dinv(degp)

    hs1 = _tc_mm_scale(x, W1e, dinv)
    g1 = _prop_kernel(hs1, src2, dst2, ew2)
    hs2 = _tc_combine_mm(g1, hs1, b1e, W2e, dinv)
    g2 = _prop_kernel(hs2, src2, dst2, ew2)
    x_encoded = _tc_combine(g2, hs2, b2e, dinv, relu=True)

    hs5 = _tc_mm_scale(x_encoded, W1s, dinv)
    g5 = _prop_kernel(hs5, src2, dst2, ew2)
    s = _tc_combine(g5, hs5, b1s, dinv, relu=True)
    struct_reconstructed = _tc_matmul_nt(s)

    hs3 = _tc_mm_scale(x_encoded, W1a, dinv)
    g3 = _prop_kernel(hs3, src2, dst2, ew2)
    hs4 = _tc_combine_mm(g3, hs3, b1a, W2a, dinv)
    g4 = _prop_kernel(hs4, src2, dst2, ew2)
    x_hat = _tc_combine(g4, hs4, b2a, dinv, relu=False)

    return (struct_reconstructed, x_hat, x_encoded)
